# Initial kernel scaffold; baseline (speedup 1.0000x reference)
import jax
import jax.numpy as jnp
from jax.experimental import pallas as pl


def kernel(x, lut):
    raise NotImplementedError("write your pallas kernel here")


# speedup vs baseline: 2.8619x; 2.8619x over previous
import functools
import math

import jax
import jax.numpy as jnp
from jax import lax
from jax.experimental import pallas as pl
from jax.experimental.pallas import tpu as pltpu
from jax.experimental.pallas import tpu_sc as plsc

_NC = 2
_NS = 16
_NW = _NC * _NS
_LANES = 16

_IDXW = 128
_K = 2
_CHUNK = _K * _IDXW


def kernel(x, lut):
    s_shape = x.shape
    B = x.size
    V, D = lut.shape
    assert B % (_NW * _CHUNK) == 0 and D % _LANES == 0
    n_chunks = B // (_NW * _CHUNK)
    rows_per_w = B // _NW
    idx_rows_per_w = rows_per_w // _IDXW
    scale = math.sqrt(float(D))

    idx2d = x.reshape(B // _IDXW, _IDXW).astype(jnp.int32)

    mesh = plsc.VectorSubcoreMesh(core_axis_name="c", subcore_axis_name="s")

    @functools.partial(
        pl.kernel,
        mesh=mesh,
        out_type=jax.ShapeDtypeStruct((B, D), jnp.float32),
        scratch_types=[
            pltpu.VMEM((2, _K, _IDXW), jnp.int32),
            pltpu.VMEM((2, _CHUNK, D), jnp.float32),
            pltpu.SemaphoreType.DMA,
            pltpu.SemaphoreType.DMA,
            pltpu.SemaphoreType.DMA,
            pltpu.SemaphoreType.DMA,
        ],
    )
    def emb(idx_hbm, table_hbm, out_hbm, idx_v, rows_v, gs0, gs1, ws0, ws1):
        gsem = (gs0, gs1)
        wsem = (ws0, ws1)
        wid = lax.axis_index("s") * _NC + lax.axis_index("c")
        idx_row0 = wid * idx_rows_per_w
        out_row0 = wid * rows_per_w

        def start_chunk(g, s):
            pltpu.sync_copy(
                idx_hbm.at[pl.ds(idx_row0 + g * _K, _K), :], idx_v.at[s])
            descs = []
            for j in range(_K):
                descs.append(pltpu.async_copy(
                    table_hbm.at[idx_v.at[s, j]],
                    rows_v.at[s, pl.ds(j * _IDXW, _IDXW), :],
                    gsem[s]))
            return descs

        pending_g = {0: start_chunk(0, 0)}
        pending_wb = [None, None]
        for g in range(n_chunks):
            s = g & 1
            if g + 1 < n_chunks:
                s2 = 1 - s
                if pending_wb[s2] is not None:
                    pending_wb[s2].wait()
                    pending_wb[s2] = None
                pending_g[g + 1] = start_chunk(g + 1, s2)
            for d in pending_g.pop(g):
                d.wait()

            def scale_row(i, carry, s=s):
                for j in range(D // _LANES):
                    sl = pl.ds(j * _LANES, _LANES)
                    rows_v[s, i, sl] = rows_v[s, i, sl] * scale
                return carry
            lax.fori_loop(0, _CHUNK, scale_row, 0)

            pending_wb[s] = pltpu.async_copy(
                rows_v.at[s],
                out_hbm.at[pl.ds(out_row0 + g * _CHUNK, _CHUNK), :],
                wsem[s])
        for s in range(2):
            if pending_wb[s] is not None:
                pending_wb[s].wait()

    out = emb(idx2d, lut)
    return out.reshape(*s_shape, D)


# speedup vs baseline: 4.8335x; 1.6889x over previous
import functools
import math

import jax
import jax.numpy as jnp
from jax import lax
from jax.experimental import pallas as pl
from jax.experimental.pallas import tpu as pltpu
from jax.experimental.pallas import tpu_sc as plsc

_NC = 2
_NS = 16
_NW = _NC * _NS
_LANES = 16

_SEQ_PER_CHUNK = 4
_GW0 = 128


def kernel(x, lut):
    n_seq, seq_len = x.shape
    B = n_seq * seq_len
    V, D = lut.shape
    chunk = _SEQ_PER_CHUNK * seq_len
    gw1 = chunk - _GW0
    assert n_seq % (_NW * _SEQ_PER_CHUNK) == 0
    assert 0 < gw1 <= 128 and gw1 % 8 == 0 and D % _LANES == 0
    seq_per_w = n_seq // _NW
    idx_per_w = seq_per_w * seq_len
    n_chunks = seq_per_w // _SEQ_PER_CHUNK
    scale = math.sqrt(float(D))

    idx_flat = x.reshape(B).astype(jnp.int32)

    mesh = plsc.VectorSubcoreMesh(core_axis_name="c", subcore_axis_name="s")

    @functools.partial(
        pl.kernel,
        mesh=mesh,
        out_type=jax.ShapeDtypeStruct((n_seq, seq_len, D), jnp.float32),
        scratch_types=[
            pltpu.VMEM((2, 1, _GW0), jnp.int32),
            pltpu.VMEM((2, 1, gw1), jnp.int32),
            pltpu.VMEM((2, chunk, D), jnp.float32),
            pltpu.VMEM((2, _SEQ_PER_CHUNK, seq_len, D), jnp.float32),
            pltpu.SemaphoreType.DMA,
            pltpu.SemaphoreType.DMA,
            pltpu.SemaphoreType.DMA,
            pltpu.SemaphoreType.DMA,
        ],
    )
    def emb(idx_hbm, table_hbm, out_hbm, idx_a, idx_b, rows_v, stage_v,
            gs0, gs1, ws0, ws1):
        gsem = (gs0, gs1)
        wsem = (ws0, ws1)
        wid = lax.axis_index("s") * _NC + lax.axis_index("c")
        idx0 = wid * idx_per_w
        seq0 = wid * seq_per_w

        def start_chunk(g, s):
            base = idx0 + g * chunk
            pltpu.sync_copy(idx_hbm.at[pl.ds(base, _GW0)], idx_a.at[s, 0])
            pltpu.sync_copy(idx_hbm.at[pl.ds(base + _GW0, gw1)],
                            idx_b.at[s, 0])
            return [
                pltpu.async_copy(table_hbm.at[idx_a.at[s, 0]],
                                 rows_v.at[s, pl.ds(0, _GW0), :], gsem[s]),
                pltpu.async_copy(table_hbm.at[idx_b.at[s, 0]],
                                 rows_v.at[s, pl.ds(_GW0, gw1), :], gsem[s]),
            ]

        pending_g = {0: start_chunk(0, 0)}
        pending_wb = [None, None]
        for g in range(n_chunks):
            s = g & 1
            if g + 1 < n_chunks:
                pending_g[g + 1] = start_chunk(g + 1, 1 - s)
            for d in pending_g.pop(g):
                d.wait()
            for d in pending_wb[s] or ():
                d.wait()
            pending_wb[s] = None

            def scale_row(r, carry, s=s):
                for i in range(_SEQ_PER_CHUNK):
                    for j in range(D // _LANES):
                        sl = pl.ds(j * _LANES, _LANES)
                        stage_v[s, i, r, sl] = (
                            rows_v[s, i * seq_len + r, sl] * scale)
                return carry
            lax.fori_loop(0, seq_len, scale_row, 0)

            wbs = []
            for i in range(_SEQ_PER_CHUNK):
                wbs.append(pltpu.async_copy(
                    stage_v.at[s, i],
                    out_hbm.at[seq0 + g * _SEQ_PER_CHUNK + i],
                    wsem[s]))
            pending_wb[s] = wbs
        for s in range(2):
            for d in pending_wb[s] or ():
                d.wait()

    return emb(idx_flat, lut)


# speedup vs baseline: 8.3321x; 1.7238x over previous
import functools
import math

import jax
import jax.numpy as jnp
from jax import lax
from jax.experimental import pallas as pl
from jax.experimental.pallas import tpu as pltpu
from jax.experimental.pallas import tpu_sc as plsc

_NC = 2
_NS = 16
_NW = _NC * _NS
_LANES = 16

_IDXW = 128
_K = 2
_CHUNK = _K * _IDXW


def kernel(x, lut):
    n_seq, seq_len = x.shape
    B = n_seq * seq_len
    V, D = lut.shape
    assert B % (_NW * _CHUNK) == 0 and D % _LANES == 0
    assert n_seq % _CHUNK == 0
    idx_per_w = B // _NW
    n_chunks = idx_per_w // _CHUNK
    chunks_per_row = n_seq // _CHUNK
    scale = math.sqrt(float(D))

    idx2d = x.T.reshape(B // _IDXW, _IDXW).astype(jnp.int32)
    idx_rows_per_w = idx_per_w // _IDXW

    mesh = plsc.VectorSubcoreMesh(core_axis_name="c", subcore_axis_name="s")

    @functools.partial(
        pl.kernel,
        mesh=mesh,
        out_type=jax.ShapeDtypeStruct((seq_len, n_seq, D), jnp.float32),
        scratch_types=[
            pltpu.VMEM((2, _K, _IDXW), jnp.int32),
            pltpu.VMEM((2, _CHUNK, D), jnp.float32),
            pltpu.SemaphoreType.DMA,
            pltpu.SemaphoreType.DMA,
            pltpu.SemaphoreType.DMA,
            pltpu.SemaphoreType.DMA,
        ],
    )
    def emb(idx_hbm, table_hbm, out_hbm, idx_v, rows_v, gs0, gs1, ws0, ws1):
        gsem = (gs0, gs1)
        wsem = (ws0, ws1)
        wid = lax.axis_index("s") * _NC + lax.axis_index("c")
        idx_row0 = wid * idx_rows_per_w
        chunk0 = wid * n_chunks

        def start_chunk(g, s):
            pltpu.sync_copy(
                idx_hbm.at[pl.ds(idx_row0 + g * _K, _K), :], idx_v.at[s])
            return [
                pltpu.async_copy(
                    table_hbm.at[idx_v.at[s, j]],
                    rows_v.at[s, pl.ds(j * _IDXW, _IDXW), :],
                    gsem[s])
                for j in range(_K)
            ]

        pending_g = {0: start_chunk(0, 0)}
        pending_wb = [None, None]
        for g in range(n_chunks):
            s = g & 1
            if g + 1 < n_chunks:
                s2 = 1 - s
                if pending_wb[s2] is not None:
                    pending_wb[s2].wait()
                    pending_wb[s2] = None
                pending_g[g + 1] = start_chunk(g + 1, s2)
            for d in pending_g.pop(g):
                d.wait()

            def scale_row(i, carry, s=s):
                for j in range(D // _LANES):
                    sl = pl.ds(j * _LANES, _LANES)
                    rows_v[s, i, sl] = rows_v[s, i, sl] * scale
                return carry
            lax.fori_loop(0, _CHUNK, scale_row, 0)

            gchunk = chunk0 + g
            pending_wb[s] = pltpu.async_copy(
                rows_v.at[s],
                out_hbm.at[gchunk // chunks_per_row,
                           pl.ds((gchunk % chunks_per_row) * _CHUNK, _CHUNK),
                           :],
                wsem[s])
        for s in range(2):
            if pending_wb[s] is not None:
                pending_wb[s].wait()

    out_t = emb(idx2d, lut)
    return jnp.transpose(out_t, (1, 0, 2))
